# baseline (speedup 1.0000x reference)
import functools

import jax
import jax.numpy as jnp
from jax import lax
from jax.experimental import pallas as pl
from jax.experimental.pallas import tpu as pltpu
from jax.experimental.pallas import tpu_sc as plsc

VOCAB = 32000
D = 1024
T = 2048
E = 8
K = 2
DFF = 4096
HOPS = 2
CAP = 640
EPS = 1e-6
NW = 32
_HI = lax.Precision.HIGHEST


def _sc_gather_rows(table, idx, n_chunks):
    B = idx.shape[0]
    b_per_w = B // NW
    c_rows = b_per_w // n_chunks
    mesh = plsc.VectorSubcoreMesh(core_axis_name="c", subcore_axis_name="s")

    @functools.partial(
        pl.kernel,
        mesh=mesh,
        out_type=jax.ShapeDtypeStruct((B, table.shape[1]), table.dtype),
        scratch_types=[
            pltpu.VMEM((b_per_w,), jnp.int32),
            pltpu.VMEM((c_rows, table.shape[1]), table.dtype),
            pltpu.VMEM((c_rows, table.shape[1]), table.dtype),
            pltpu.SemaphoreType.DMA,
            pltpu.SemaphoreType.DMA,
            pltpu.SemaphoreType.DMA,
            pltpu.SemaphoreType.DMA,
        ],
    )
    def k(table_hbm, idx_hbm, out_hbm, idx_v, rows0, rows1, gs0, gs1, ws0, ws1):
        wid = lax.axis_index("s") * 2 + lax.axis_index("c")
        base = wid * b_per_w
        pltpu.sync_copy(idx_hbm.at[pl.ds(base, b_per_w)], idx_v)
        bufs, gsems, wsems = (rows0, rows1), (gs0, gs1), (ws0, ws1)
        g = [None, None]
        wb = [None, None]

        def issue(c):
            b = c & 1
            if wb[b] is not None:
                wb[b].wait()
                wb[b] = None
            g[b] = pltpu.async_copy(
                table_hbm.at[idx_v.at[pl.ds(c * c_rows, c_rows)]],
                bufs[b], gsems[b],
            )

        issue(0)
        for c in range(n_chunks):
            b = c & 1
            if c + 1 < n_chunks:
                issue(c + 1)
            g[b].wait()
            wb[b] = pltpu.async_copy(
                bufs[b], out_hbm.at[pl.ds(base + c * c_rows, c_rows)], wsems[b]
            )
        for b in range(2):
            if wb[b] is not None:
                wb[b].wait()

    return k(table, idx)


def _routing_body(h_ref, wr_ref, tok_ref, slot_ref, wgt_ref, rho_ref):
    h = h_ref[...]
    wr = wr_ref[...]
    logits = jnp.dot(h, wr, preferred_element_type=jnp.float32, precision=_HI)
    m = jnp.max(logits, axis=1, keepdims=True)
    ex = jnp.exp(logits - m)
    probs = ex / jnp.sum(ex, axis=1, keepdims=True)

    col = lax.broadcasted_iota(jnp.int32, logits.shape, 1)
    is1 = logits >= m
    idx1 = jnp.min(jnp.where(is1, col, E + 1), axis=1, keepdims=True)
    neg = jnp.where(col == idx1, -jnp.inf, logits)
    m2 = jnp.max(neg, axis=1, keepdims=True)
    idx2 = jnp.min(jnp.where(neg >= m2, col, E + 1), axis=1, keepdims=True)

    cole = lax.broadcasted_iota(jnp.int32, (T, E), 1)
    mask = (cole == idx1) | (cole == idx2)
    maskf = mask.astype(jnp.float32)

    ri = lax.broadcasted_iota(jnp.int32, (128, 128), 0)
    ci = lax.broadcasted_iota(jnp.int32, (128, 128), 1)
    tril = (ci <= ri).astype(jnp.float32)
    blocks = []
    carry = jnp.zeros((1, E), jnp.float32)
    for b in range(T // 128):
        mb = maskf[b * 128 : (b + 1) * 128, :]
        blocks.append(
            jnp.dot(tril, mb, preferred_element_type=jnp.float32, precision=_HI)
            + carry
        )
        carry = carry + jnp.sum(mb, axis=0, keepdims=True)
    pos = jnp.concatenate(blocks, axis=0)

    keptf = jnp.where(pos <= float(CAP), maskf, 0.0)
    wgt_e = probs[:, :E] * keptf
    rho_ref[...] = jnp.sum(wgt_e, axis=1, keepdims=True)

    outs_w, outs_s = [], []
    for idxk in (idx1, idx2):
        self_f = (cole == idxk).astype(jnp.float32)
        w_k = jnp.sum(wgt_e * self_f, axis=1, keepdims=True)
        pos_k = jnp.sum(pos * self_f, axis=1, keepdims=True)
        kept_k = jnp.sum(keptf * self_f, axis=1, keepdims=True)
        slot_f = idxk.astype(jnp.float32) * float(CAP) + pos_k - 1.0
        slot_k = jnp.where(kept_k > 0.0, slot_f, 0.0).astype(jnp.int32)
        outs_w.append(w_k)
        outs_s.append(slot_k)
    wgt_ref[...] = jnp.concatenate(outs_w, axis=1)
    slot_ref[...] = jnp.concatenate(outs_s, axis=1)

    t_col = lax.broadcasted_iota(jnp.int32, (T, 1), 0).astype(jnp.float32)
    s_row = lax.broadcasted_iota(jnp.int32, (1, CAP), 1).astype(jnp.float32) + 1.0
    rows = []
    for e in range(E):
        ind = jnp.where(
            (pos[:, e : e + 1] == s_row) & (keptf[:, e : e + 1] > 0.0), t_col, 0.0
        )
        rows.append(jnp.sum(ind, axis=0, keepdims=True))
    tok_ref[...] = jnp.concatenate(rows, axis=0).astype(jnp.int32)


def _routing(h, wr):
    return pl.pallas_call(
        _routing_body,
        out_shape=(
            jax.ShapeDtypeStruct((E, CAP), jnp.int32),
            jax.ShapeDtypeStruct((T, K), jnp.int32),
            jax.ShapeDtypeStruct((T, K), jnp.float32),
            jax.ShapeDtypeStruct((T, 1), jnp.float32),
        ),
    )(h, wr)


def _mlp_body(x_ref, w1_ref, w2_ref, y_ref):
    f = pl.program_id(1)
    hidden = jnp.dot(x_ref[0].astype(jnp.bfloat16), w1_ref[0].astype(jnp.bfloat16),
                     preferred_element_type=jnp.float32)
    hidden = jax.nn.gelu(hidden)
    y = jnp.dot(hidden.astype(jnp.bfloat16), w2_ref[0].astype(jnp.bfloat16),
                preferred_element_type=jnp.float32)

    @pl.when(f == 0)
    def _():
        y_ref[0] = y

    @pl.when(f != 0)
    def _():
        y_ref[0] += y


def _mlp(x_buf, w1, w2):
    ft = 4
    return pl.pallas_call(
        _mlp_body,
        grid=(E, ft),
        in_specs=[
            pl.BlockSpec((1, CAP, D), lambda e, f: (e, 0, 0)),
            pl.BlockSpec((1, D, DFF // ft), lambda e, f: (e, 0, f)),
            pl.BlockSpec((1, DFF // ft, D), lambda e, f: (e, f, 0)),
        ],
        out_specs=pl.BlockSpec((1, CAP, D), lambda e, f: (e, 0, 0)),
        out_shape=jax.ShapeDtypeStruct((E, CAP, D), jnp.float32),
    )(x_buf, w1, w2)


def _combine_body(h_ref, g_ref, wgt_ref, rho_ref, o_ref):
    h = h_ref[...]
    w0 = wgt_ref[:, 0:1]
    w1 = wgt_ref[:, 1:2]
    o_ref[...] = (
        h * (1.0 - rho_ref[...]) + w0 * g_ref[:, 0, :] + w1 * g_ref[:, 1, :]
    )


def _combine(h, g, wgt, rho):
    tt = 4
    return pl.pallas_call(
        _combine_body,
        grid=(tt,),
        in_specs=[
            pl.BlockSpec((T // tt, D), lambda i: (i, 0)),
            pl.BlockSpec((T // tt, K, D), lambda i: (i, 0, 0)),
            pl.BlockSpec((T // tt, K), lambda i: (i, 0)),
            pl.BlockSpec((T // tt, 1), lambda i: (i, 0)),
        ],
        out_specs=pl.BlockSpec((T // tt, D), lambda i: (i, 0)),
        out_shape=jax.ShapeDtypeStruct((T, D), jnp.float32),
    )(h, g, wgt, rho)


def _unembed_body(h_ref, wln_ref, we_ref, o_ref):
    h = h_ref[...]
    rms = jnp.sqrt(jnp.mean(h * h, axis=1, keepdims=True) + EPS)
    hn = (h / rms) * wln_ref[...]
    o_ref[...] = lax.dot_general(
        hn.astype(jnp.bfloat16),
        we_ref[...].astype(jnp.bfloat16),
        (((1,), (1,)), ((), ())),
        preferred_element_type=jnp.float32,
    )


def _unembed(h, wln, we):
    vt = 1280
    return pl.pallas_call(
        _unembed_body,
        grid=(VOCAB // vt,),
        in_specs=[
            pl.BlockSpec((T, D), lambda v: (0, 0)),
            pl.BlockSpec((1, D), lambda v: (0, 0)),
            pl.BlockSpec((vt, D), lambda v: (v, 0)),
        ],
        out_specs=pl.BlockSpec((T, vt), lambda v: (0, v)),
        out_shape=jax.ShapeDtypeStruct((T, VOCAB), jnp.float32),
    )(h, wln, we)


def kernel(ids_t, W_embed, w_ln, W_router, W1, W2):
    h = _sc_gather_rows(W_embed, ids_t.astype(jnp.int32), n_chunks=2)
    for hop in range(HOPS):
        tok_idx, slot, wgt, rho = _routing(h, W_router[hop])
        x_buf = _sc_gather_rows(h, tok_idx.reshape(-1), n_chunks=4)
        y = _mlp(x_buf.reshape(E, CAP, D), W1[hop], W2[hop])
        g = _sc_gather_rows(y.reshape(E * CAP, D), slot.reshape(-1), n_chunks=4)
        h = _combine(h, g.reshape(T, K, D), wgt, rho)
    return _unembed(h, w_ln.reshape(1, D), W_embed)


# speedup vs baseline: 1.2397x; 1.2397x over previous
import functools

import jax
import jax.numpy as jnp
from jax import lax
from jax.experimental import pallas as pl
from jax.experimental.pallas import tpu as pltpu
from jax.experimental.pallas import tpu_sc as plsc

VOCAB = 32000
D = 1024
T = 2048
E = 8
K = 2
DFF = 4096
HOPS = 2
CAP = 640
EPS = 1e-6
NW = 32
_HI = lax.Precision.HIGHEST


def _sc_gather_rows(table, idx, n_chunks):
    B = idx.shape[0]
    b_per_w = B // NW
    c_rows = b_per_w // n_chunks
    mesh = plsc.VectorSubcoreMesh(core_axis_name="c", subcore_axis_name="s")

    @functools.partial(
        pl.kernel,
        mesh=mesh,
        out_type=jax.ShapeDtypeStruct((B, table.shape[1]), table.dtype),
        scratch_types=[
            pltpu.VMEM((b_per_w,), jnp.int32),
            pltpu.VMEM((c_rows, table.shape[1]), table.dtype),
            pltpu.VMEM((c_rows, table.shape[1]), table.dtype),
            pltpu.SemaphoreType.DMA,
            pltpu.SemaphoreType.DMA,
            pltpu.SemaphoreType.DMA,
            pltpu.SemaphoreType.DMA,
        ],
    )
    def k(table_hbm, idx_hbm, out_hbm, idx_v, rows0, rows1, gs0, gs1, ws0, ws1):
        wid = lax.axis_index("s") * 2 + lax.axis_index("c")
        base = wid * b_per_w
        pltpu.sync_copy(idx_hbm.at[pl.ds(base, b_per_w)], idx_v)
        bufs, gsems, wsems = (rows0, rows1), (gs0, gs1), (ws0, ws1)
        g = [None, None]
        wb = [None, None]

        def issue(c):
            b = c & 1
            if wb[b] is not None:
                wb[b].wait()
                wb[b] = None
            g[b] = pltpu.async_copy(
                table_hbm.at[idx_v.at[pl.ds(c * c_rows, c_rows)]],
                bufs[b], gsems[b],
            )

        issue(0)
        for c in range(n_chunks):
            b = c & 1
            if c + 1 < n_chunks:
                issue(c + 1)
            g[b].wait()
            wb[b] = pltpu.async_copy(
                bufs[b], out_hbm.at[pl.ds(base + c * c_rows, c_rows)], wsems[b]
            )
        for b in range(2):
            if wb[b] is not None:
                wb[b].wait()

    return k(table, idx)


def _routing_body(h_ref, wr_ref, tok_ref, slot_ref, wgt_ref, rho_ref):
    h = h_ref[...]
    wr = wr_ref[...]
    logits = jnp.dot(h, wr, preferred_element_type=jnp.float32, precision=_HI)
    m = jnp.max(logits, axis=1, keepdims=True)
    ex = jnp.exp(logits - m)
    probs = ex / jnp.sum(ex, axis=1, keepdims=True)

    col = lax.broadcasted_iota(jnp.int32, logits.shape, 1)
    is1 = logits >= m
    idx1 = jnp.min(jnp.where(is1, col, E + 1), axis=1, keepdims=True)
    neg = jnp.where(col == idx1, -jnp.inf, logits)
    m2 = jnp.max(neg, axis=1, keepdims=True)
    idx2 = jnp.min(jnp.where(neg >= m2, col, E + 1), axis=1, keepdims=True)

    cole = lax.broadcasted_iota(jnp.int32, (T, E), 1)
    mask = (cole == idx1) | (cole == idx2)
    maskf = mask.astype(jnp.float32)

    ri = lax.broadcasted_iota(jnp.int32, (128, 128), 0)
    ci = lax.broadcasted_iota(jnp.int32, (128, 128), 1)
    tril = (ci <= ri).astype(jnp.float32)
    blocks = []
    carry = jnp.zeros((1, E), jnp.float32)
    for b in range(T // 128):
        mb = maskf[b * 128 : (b + 1) * 128, :]
        blocks.append(
            jnp.dot(tril, mb, preferred_element_type=jnp.float32, precision=_HI)
            + carry
        )
        carry = carry + jnp.sum(mb, axis=0, keepdims=True)
    pos = jnp.concatenate(blocks, axis=0)

    keptf = jnp.where(pos <= float(CAP), maskf, 0.0)
    wgt_e = probs[:, :E] * keptf
    rho_ref[...] = jnp.sum(wgt_e, axis=1, keepdims=True)

    t_colf = lax.broadcasted_iota(jnp.int32, (T, 1), 0).astype(jnp.float32)
    outs_w, outs_s = [], []
    for idxk in (idx1, idx2):
        self_f = (cole == idxk).astype(jnp.float32)
        w_k = jnp.sum(wgt_e * self_f, axis=1, keepdims=True)
        pos_k = jnp.sum(pos * self_f, axis=1, keepdims=True)
        kept_k = jnp.sum(keptf * self_f, axis=1, keepdims=True)
        fb = 2.0 * t_colf + (0.0 if idxk is idx1 else 1.0)
        slot_f = idxk.astype(jnp.float32) * float(CAP) + pos_k - 1.0
        slot_k = jnp.where(kept_k > 0.0, slot_f, fb).astype(jnp.int32)
        outs_w.append(w_k)
        outs_s.append(slot_k)
    wgt_ref[...] = jnp.concatenate(outs_w, axis=1)
    slot_ref[...] = jnp.concatenate(outs_s, axis=1)

    s_row = lax.broadcasted_iota(jnp.int32, (1, CAP), 1).astype(jnp.float32) + 1.0
    rows = []
    for e in range(E):
        sel = (pos[:, e : e + 1] == s_row) & (keptf[:, e : e + 1] > 0.0)
        tok_e = jnp.sum(jnp.where(sel, t_colf, 0.0), axis=0, keepdims=True)
        cnt_e = jnp.sum(keptf[:, e : e + 1], axis=0, keepdims=True)
        flat_e = float(e) * float(CAP) + (s_row - 1.0)
        fb_e = flat_e - jnp.floor(flat_e / float(T)) * float(T)
        rows.append(jnp.where(s_row - 1.0 < cnt_e, tok_e, fb_e))
    tok_ref[...] = jnp.concatenate(rows, axis=0).astype(jnp.int32)


def _routing(h, wr):
    return pl.pallas_call(
        _routing_body,
        out_shape=(
            jax.ShapeDtypeStruct((E, CAP), jnp.int32),
            jax.ShapeDtypeStruct((T, K), jnp.int32),
            jax.ShapeDtypeStruct((T, K), jnp.float32),
            jax.ShapeDtypeStruct((T, 1), jnp.float32),
        ),
    )(h, wr)


def _mlp_body(x_ref, w1_ref, w2_ref, y_ref):
    f = pl.program_id(1)
    hidden = jnp.dot(x_ref[0].astype(jnp.bfloat16), w1_ref[0].astype(jnp.bfloat16),
                     preferred_element_type=jnp.float32)
    hidden = jax.nn.gelu(hidden)
    y = jnp.dot(hidden.astype(jnp.bfloat16), w2_ref[0].astype(jnp.bfloat16),
                preferred_element_type=jnp.float32)

    @pl.when(f == 0)
    def _():
        y_ref[0] = y

    @pl.when(f != 0)
    def _():
        y_ref[0] += y


def _mlp(x_buf, w1, w2):
    ft = 4
    return pl.pallas_call(
        _mlp_body,
        grid=(E, ft),
        in_specs=[
            pl.BlockSpec((1, CAP, D), lambda e, f: (e, 0, 0)),
            pl.BlockSpec((1, D, DFF // ft), lambda e, f: (e, 0, f)),
            pl.BlockSpec((1, DFF // ft, D), lambda e, f: (e, f, 0)),
        ],
        out_specs=pl.BlockSpec((1, CAP, D), lambda e, f: (e, 0, 0)),
        out_shape=jax.ShapeDtypeStruct((E, CAP, D), jnp.float32),
    )(x_buf, w1, w2)


def _combine_body(h_ref, g_ref, wgt_ref, rho_ref, o_ref):
    h = h_ref[...]
    w0 = wgt_ref[:, 0:1]
    w1 = wgt_ref[:, 1:2]
    o_ref[...] = (
        h * (1.0 - rho_ref[...]) + w0 * g_ref[:, 0, :] + w1 * g_ref[:, 1, :]
    )


def _combine(h, g, wgt, rho):
    tt = 4
    return pl.pallas_call(
        _combine_body,
        grid=(tt,),
        in_specs=[
            pl.BlockSpec((T // tt, D), lambda i: (i, 0)),
            pl.BlockSpec((T // tt, K, D), lambda i: (i, 0, 0)),
            pl.BlockSpec((T // tt, K), lambda i: (i, 0)),
            pl.BlockSpec((T // tt, 1), lambda i: (i, 0)),
        ],
        out_specs=pl.BlockSpec((T // tt, D), lambda i: (i, 0)),
        out_shape=jax.ShapeDtypeStruct((T, D), jnp.float32),
    )(h, g, wgt, rho)


def _unembed_body(h_ref, wln_ref, we_ref, o_ref):
    h = h_ref[...]
    rms = jnp.sqrt(jnp.mean(h * h, axis=1, keepdims=True) + EPS)
    hn = (h / rms) * wln_ref[...]
    o_ref[...] = lax.dot_general(
        hn.astype(jnp.bfloat16),
        we_ref[...].astype(jnp.bfloat16),
        (((1,), (1,)), ((), ())),
        preferred_element_type=jnp.float32,
    )


def _unembed(h, wln, we):
    vt = 1280
    return pl.pallas_call(
        _unembed_body,
        grid=(VOCAB // vt,),
        in_specs=[
            pl.BlockSpec((T, D), lambda v: (0, 0)),
            pl.BlockSpec((1, D), lambda v: (0, 0)),
            pl.BlockSpec((vt, D), lambda v: (v, 0)),
        ],
        out_specs=pl.BlockSpec((T, vt), lambda v: (0, v)),
        out_shape=jax.ShapeDtypeStruct((T, VOCAB), jnp.float32),
    )(h, wln, we)


def kernel(ids_t, W_embed, w_ln, W_router, W1, W2):
    h = _sc_gather_rows(W_embed, ids_t.astype(jnp.int32), n_chunks=2)
    for hop in range(HOPS):
        tok_idx, slot, wgt, rho = _routing(h, W_router[hop])
        x_buf = _sc_gather_rows(h, tok_idx.reshape(-1), n_chunks=4)
        y = _mlp(x_buf.reshape(E, CAP, D), W1[hop], W2[hop])
        g = _sc_gather_rows(y.reshape(E * CAP, D), slot.reshape(-1), n_chunks=4)
        h = _combine(h, g.reshape(T, K, D), wgt, rho)
    return _unembed(h, w_ln.reshape(1, D), W_embed)


# speedup vs baseline: 1.7911x; 1.4448x over previous
import functools

import jax
import jax.numpy as jnp
from jax import lax
from jax.experimental import pallas as pl
from jax.experimental.pallas import tpu as pltpu
from jax.experimental.pallas import tpu_sc as plsc

VOCAB = 32000
D = 1024
T = 2048
E = 8
K = 2
DFF = 4096
HOPS = 2
CAP = 640
EPS = 1e-6
NW = 32
_HI = lax.Precision.HIGHEST


def _sc_gather_rows(table, idx, n_chunks):
    B = idx.shape[0]
    b_per_w = B // NW
    c_rows = b_per_w // n_chunks
    mesh = plsc.VectorSubcoreMesh(core_axis_name="c", subcore_axis_name="s")

    @functools.partial(
        pl.kernel,
        mesh=mesh,
        out_type=jax.ShapeDtypeStruct((B, table.shape[1]), table.dtype),
        scratch_types=[
            pltpu.VMEM((b_per_w,), jnp.int32),
            pltpu.VMEM((c_rows, table.shape[1]), table.dtype),
            pltpu.VMEM((c_rows, table.shape[1]), table.dtype),
            pltpu.SemaphoreType.DMA,
            pltpu.SemaphoreType.DMA,
            pltpu.SemaphoreType.DMA,
            pltpu.SemaphoreType.DMA,
        ],
    )
    def k(table_hbm, idx_hbm, out_hbm, idx_v, rows0, rows1, gs0, gs1, ws0, ws1):
        wid = lax.axis_index("s") * 2 + lax.axis_index("c")
        base = wid * b_per_w
        pltpu.sync_copy(idx_hbm.at[pl.ds(base, b_per_w)], idx_v)
        bufs, gsems, wsems = (rows0, rows1), (gs0, gs1), (ws0, ws1)
        g = [None, None]
        wb = [None, None]

        def issue(c):
            b = c & 1
            if wb[b] is not None:
                wb[b].wait()
                wb[b] = None
            g[b] = pltpu.async_copy(
                table_hbm.at[idx_v.at[pl.ds(c * c_rows, c_rows)]],
                bufs[b], gsems[b],
            )

        issue(0)
        for c in range(n_chunks):
            b = c & 1
            if c + 1 < n_chunks:
                issue(c + 1)
            g[b].wait()
            wb[b] = pltpu.async_copy(
                bufs[b], out_hbm.at[pl.ds(base + c * c_rows, c_rows)], wsems[b]
            )
        for b in range(2):
            if wb[b] is not None:
                wb[b].wait()

    return k(table, idx)


def _routing_body(h_ref, wr_ref, tok_ref, slot_ref, wgt_ref, rho_ref):
    h = h_ref[...]
    wr = wr_ref[0]
    logits = jnp.dot(h, wr, preferred_element_type=jnp.float32, precision=_HI)
    m = jnp.max(logits, axis=1, keepdims=True)
    ex = jnp.exp(logits - m)
    probs = ex / jnp.sum(ex, axis=1, keepdims=True)

    col = lax.broadcasted_iota(jnp.int32, logits.shape, 1)
    is1 = logits >= m
    idx1 = jnp.min(jnp.where(is1, col, E + 1), axis=1, keepdims=True)
    neg = jnp.where(col == idx1, -jnp.inf, logits)
    m2 = jnp.max(neg, axis=1, keepdims=True)
    idx2 = jnp.min(jnp.where(neg >= m2, col, E + 1), axis=1, keepdims=True)

    cole = lax.broadcasted_iota(jnp.int32, (T, E), 1)
    mask = (cole == idx1) | (cole == idx2)
    maskf = mask.astype(jnp.float32)

    ri = lax.broadcasted_iota(jnp.int32, (128, 128), 0)
    ci = lax.broadcasted_iota(jnp.int32, (128, 128), 1)
    tril = (ci <= ri).astype(jnp.float32)
    blocks = []
    carry = jnp.zeros((1, E), jnp.float32)
    for b in range(T // 128):
        mb = maskf[b * 128 : (b + 1) * 128, :]
        blocks.append(
            jnp.dot(tril, mb, preferred_element_type=jnp.float32, precision=_HI)
            + carry
        )
        carry = carry + jnp.sum(mb, axis=0, keepdims=True)
    pos = jnp.concatenate(blocks, axis=0)

    keptf = jnp.where(pos <= float(CAP), maskf, 0.0)
    wgt_e = probs[:, :E] * keptf
    rho_ref[...] = jnp.sum(wgt_e, axis=1, keepdims=True)

    t_colf = lax.broadcasted_iota(jnp.int32, (T, 1), 0).astype(jnp.float32)
    outs_w, outs_s = [], []
    for idxk in (idx1, idx2):
        self_f = (cole == idxk).astype(jnp.float32)
        w_k = jnp.sum(wgt_e * self_f, axis=1, keepdims=True)
        pos_k = jnp.sum(pos * self_f, axis=1, keepdims=True)
        kept_k = jnp.sum(keptf * self_f, axis=1, keepdims=True)
        fb = 2.0 * t_colf + (0.0 if idxk is idx1 else 1.0)
        slot_f = idxk.astype(jnp.float32) * float(CAP) + pos_k - 1.0
        slot_k = jnp.where(kept_k > 0.0, slot_f, fb).astype(jnp.int32)
        outs_w.append(w_k)
        outs_s.append(slot_k)
    wgt_ref[...] = jnp.concatenate(outs_w, axis=1)
    slot_ref[...] = jnp.concatenate(outs_s, axis=1)

    s_row = lax.broadcasted_iota(jnp.int32, (1, CAP), 1).astype(jnp.float32) + 1.0
    rows = []
    for e in range(E):
        sel = (pos[:, e : e + 1] == s_row) & (keptf[:, e : e + 1] > 0.0)
        tok_e = jnp.sum(jnp.where(sel, t_colf, 0.0), axis=0, keepdims=True)
        cnt_e = jnp.sum(keptf[:, e : e + 1], axis=0, keepdims=True)
        flat_e = float(e) * float(CAP) + (s_row - 1.0)
        fb_e = flat_e - jnp.floor(flat_e / float(T)) * float(T)
        rows.append(jnp.where(s_row - 1.0 < cnt_e, tok_e, fb_e))
    tok_ref[...] = jnp.concatenate(rows, axis=0).astype(jnp.int32)


def _routing(hop, h, w_router):
    return pl.pallas_call(
        _routing_body,
        grid=(1,),
        in_specs=[
            pl.BlockSpec((T, D), lambda i: (0, 0)),
            pl.BlockSpec((1, D, E + 1), lambda i: (hop, 0, 0)),
        ],
        out_specs=(
            pl.BlockSpec((E, CAP), lambda i: (0, 0)),
            pl.BlockSpec((T, K), lambda i: (0, 0)),
            pl.BlockSpec((T, K), lambda i: (0, 0)),
            pl.BlockSpec((T, 1), lambda i: (0, 0)),
        ),
        out_shape=(
            jax.ShapeDtypeStruct((E, CAP), jnp.int32),
            jax.ShapeDtypeStruct((T, K), jnp.int32),
            jax.ShapeDtypeStruct((T, K), jnp.float32),
            jax.ShapeDtypeStruct((T, 1), jnp.float32),
        ),
    )(h, w_router)


def _mlp_body(x_ref, w1_ref, w2_ref, y_ref):
    f = pl.program_id(1)
    hidden = jnp.dot(x_ref[0].astype(jnp.bfloat16),
                     w1_ref[0, 0].astype(jnp.bfloat16),
                     preferred_element_type=jnp.float32)
    hidden = jax.nn.gelu(hidden)
    y = jnp.dot(hidden.astype(jnp.bfloat16), w2_ref[0, 0].astype(jnp.bfloat16),
                preferred_element_type=jnp.float32)

    @pl.when(f == 0)
    def _():
        y_ref[0] = y

    @pl.when(f != 0)
    def _():
        y_ref[0] += y


def _mlp(hop, e_base, x_buf, w1, w2):
    n_e = x_buf.shape[0]
    ft = 4
    return pl.pallas_call(
        _mlp_body,
        grid=(n_e, ft),
        in_specs=[
            pl.BlockSpec((1, CAP, D), lambda e, f: (e, 0, 0)),
            pl.BlockSpec((1, 1, D, DFF // ft),
                         lambda e, f: (hop, e_base + e, 0, f)),
            pl.BlockSpec((1, 1, DFF // ft, D),
                         lambda e, f: (hop, e_base + e, f, 0)),
        ],
        out_specs=pl.BlockSpec((1, CAP, D), lambda e, f: (e, 0, 0)),
        out_shape=jax.ShapeDtypeStruct((n_e, CAP, D), jnp.float32),
    )(x_buf, w1, w2)


def _combine_body(h_ref, g_ref, wgt_ref, rho_ref, o_ref):
    h = h_ref[...]
    w0 = wgt_ref[:, 0:1]
    w1 = wgt_ref[:, 1:2]
    o_ref[...] = (
        h * (1.0 - rho_ref[...]) + w0 * g_ref[:, 0, :] + w1 * g_ref[:, 1, :]
    )


def _combine(h, g, wgt, rho):
    tt = 4
    return pl.pallas_call(
        _combine_body,
        grid=(tt,),
        in_specs=[
            pl.BlockSpec((T // tt, D), lambda i: (i, 0)),
            pl.BlockSpec((T // tt, K, D), lambda i: (i, 0, 0)),
            pl.BlockSpec((T // tt, K), lambda i: (i, 0)),
            pl.BlockSpec((T // tt, 1), lambda i: (i, 0)),
        ],
        out_specs=pl.BlockSpec((T // tt, D), lambda i: (i, 0)),
        out_shape=jax.ShapeDtypeStruct((T, D), jnp.float32),
    )(h, g, wgt, rho)


def _unembed_body(h_ref, wln_ref, we_ref, o_ref):
    h = h_ref[...]
    rms = jnp.sqrt(jnp.mean(h * h, axis=1, keepdims=True) + EPS)
    hn = (h / rms) * wln_ref[...]
    o_ref[...] = lax.dot_general(
        hn.astype(jnp.bfloat16),
        we_ref[...].astype(jnp.bfloat16),
        (((1,), (1,)), ((), ())),
        preferred_element_type=jnp.float32,
    )


def _unembed(h, wln, we):
    vt = 1280
    return pl.pallas_call(
        _unembed_body,
        grid=(VOCAB // vt,),
        in_specs=[
            pl.BlockSpec((T, D), lambda v: (0, 0)),
            pl.BlockSpec((1, D), lambda v: (0, 0)),
            pl.BlockSpec((vt, D), lambda v: (v, 0)),
        ],
        out_specs=pl.BlockSpec((T, vt), lambda v: (0, v)),
        out_shape=jax.ShapeDtypeStruct((T, VOCAB), jnp.float32),
    )(h, wln, we)


def kernel(ids_t, W_embed, w_ln, W_router, W1, W2):
    h = _sc_gather_rows(W_embed, ids_t.astype(jnp.int32), n_chunks=2)
    eh = E // 2
    for hop in range(HOPS):
        tok_idx, slot, wgt, rho = _routing(hop, h, W_router)
        xb0 = _sc_gather_rows(h, tok_idx[:eh].reshape(-1), n_chunks=2)
        xb1 = _sc_gather_rows(h, tok_idx[eh:].reshape(-1), n_chunks=2)
        y0 = _mlp(hop, 0, xb0.reshape(eh, CAP, D), W1, W2)
        y1 = _mlp(hop, eh, xb1.reshape(eh, CAP, D), W1, W2)
        y = jnp.concatenate([y0, y1], axis=0)
        g = _sc_gather_rows(y.reshape(E * CAP, D), slot.reshape(-1), n_chunks=4)
        h = _combine(h, g.reshape(T, K, D), wgt, rho)
    return _unembed(h, w_ln.reshape(1, D), W_embed)


# speedup vs baseline: 1.8271x; 1.0201x over previous
import functools

import jax
import jax.numpy as jnp
from jax import lax
from jax.experimental import pallas as pl
from jax.experimental.pallas import tpu as pltpu
from jax.experimental.pallas import tpu_sc as plsc

VOCAB = 32000
D = 1024
T = 2048
E = 8
K = 2
DFF = 4096
HOPS = 2
CAP = 640
EPS = 1e-6
NW = 32
_HI = lax.Precision.HIGHEST


def _sc_gather_rows(table, idx, n_chunks):
    B = idx.shape[0]
    b_per_w = B // NW
    c_rows = b_per_w // n_chunks
    mesh = plsc.VectorSubcoreMesh(core_axis_name="c", subcore_axis_name="s")

    @functools.partial(
        pl.kernel,
        mesh=mesh,
        out_type=jax.ShapeDtypeStruct((B, table.shape[1]), table.dtype),
        scratch_types=[
            pltpu.VMEM((b_per_w,), jnp.int32),
            pltpu.VMEM((c_rows, table.shape[1]), table.dtype),
            pltpu.VMEM((c_rows, table.shape[1]), table.dtype),
            pltpu.SemaphoreType.DMA,
            pltpu.SemaphoreType.DMA,
            pltpu.SemaphoreType.DMA,
            pltpu.SemaphoreType.DMA,
        ],
    )
    def k(table_hbm, idx_hbm, out_hbm, idx_v, rows0, rows1, gs0, gs1, ws0, ws1):
        wid = lax.axis_index("s") * 2 + lax.axis_index("c")
        base = wid * b_per_w
        pltpu.sync_copy(idx_hbm.at[pl.ds(base, b_per_w)], idx_v)
        bufs, gsems, wsems = (rows0, rows1), (gs0, gs1), (ws0, ws1)
        g = [None, None]
        wb = [None, None]

        def issue(c):
            b = c & 1
            if wb[b] is not None:
                wb[b].wait()
                wb[b] = None
            g[b] = pltpu.async_copy(
                table_hbm.at[idx_v.at[pl.ds(c * c_rows, c_rows)]],
                bufs[b], gsems[b],
            )

        issue(0)
        for c in range(n_chunks):
            b = c & 1
            if c + 1 < n_chunks:
                issue(c + 1)
            g[b].wait()
            wb[b] = pltpu.async_copy(
                bufs[b], out_hbm.at[pl.ds(base + c * c_rows, c_rows)], wsems[b]
            )
        for b in range(2):
            if wb[b] is not None:
                wb[b].wait()

    return k(table, idx)


def _routing_body(h_ref, wr_ref, tok_ref, slot_ref, wgt_ref, rho_ref):
    h = h_ref[...]
    wr = wr_ref[0]
    logits = jnp.dot(h, wr, preferred_element_type=jnp.float32, precision=_HI)
    m = jnp.max(logits, axis=1, keepdims=True)
    ex = jnp.exp(logits - m)
    probs = ex / jnp.sum(ex, axis=1, keepdims=True)

    col = lax.broadcasted_iota(jnp.int32, logits.shape, 1)
    is1 = logits >= m
    idx1 = jnp.min(jnp.where(is1, col, E + 1), axis=1, keepdims=True)
    neg = jnp.where(col == idx1, -jnp.inf, logits)
    m2 = jnp.max(neg, axis=1, keepdims=True)
    idx2 = jnp.min(jnp.where(neg >= m2, col, E + 1), axis=1, keepdims=True)

    cole = lax.broadcasted_iota(jnp.int32, (T, E), 1)
    mask = (cole == idx1) | (cole == idx2)
    maskf = mask.astype(jnp.float32)

    ri = lax.broadcasted_iota(jnp.int32, (128, 128), 0)
    ci = lax.broadcasted_iota(jnp.int32, (128, 128), 1)
    tril = (ci <= ri).astype(jnp.bfloat16)
    blocks = []
    carry = jnp.zeros((1, E), jnp.float32)
    for b in range(T // 128):
        mb = maskf[b * 128 : (b + 1) * 128, :]
        blocks.append(
            jnp.dot(tril, mb.astype(jnp.bfloat16),
                    preferred_element_type=jnp.float32)
            + carry
        )
        carry = carry + jnp.sum(mb, axis=0, keepdims=True)
    pos = jnp.concatenate(blocks, axis=0)

    keptf = jnp.where(pos <= float(CAP), maskf, 0.0)
    wgt_e = probs[:, :E] * keptf
    rho_ref[...] = jnp.sum(wgt_e, axis=1, keepdims=True)

    t_colf = lax.broadcasted_iota(jnp.int32, (T, 1), 0).astype(jnp.float32)
    outs_w, outs_s = [], []
    for idxk in (idx1, idx2):
        self_f = (cole == idxk).astype(jnp.float32)
        w_k = jnp.sum(wgt_e * self_f, axis=1, keepdims=True)
        pos_k = jnp.sum(pos * self_f, axis=1, keepdims=True)
        kept_k = jnp.sum(keptf * self_f, axis=1, keepdims=True)
        fb = 2.0 * t_colf + (0.0 if idxk is idx1 else 1.0)
        slot_f = idxk.astype(jnp.float32) * float(CAP) + pos_k - 1.0
        slot_k = jnp.where(kept_k > 0.0, slot_f, fb).astype(jnp.int32)
        outs_w.append(w_k)
        outs_s.append(slot_k)
    wgt_ref[...] = jnp.concatenate(outs_w, axis=1)
    slot_ref[...] = jnp.concatenate(outs_s, axis=1)

    s_row = lax.broadcasted_iota(jnp.int32, (1, CAP), 1).astype(jnp.float32) + 1.0
    rows = []
    for e in range(E):
        sel = (pos[:, e : e + 1] == s_row) & (keptf[:, e : e + 1] > 0.0)
        tok_e = jnp.sum(jnp.where(sel, t_colf, 0.0), axis=0, keepdims=True)
        cnt_e = jnp.sum(keptf[:, e : e + 1], axis=0, keepdims=True)
        flat_e = float(e) * float(CAP) + (s_row - 1.0)
        fb_e = flat_e - jnp.floor(flat_e / float(T)) * float(T)
        rows.append(jnp.where(s_row - 1.0 < cnt_e, tok_e, fb_e))
    tok_ref[...] = jnp.concatenate(rows, axis=0).astype(jnp.int32)


def _routing(hop, h, w_router):
    return pl.pallas_call(
        _routing_body,
        grid=(1,),
        in_specs=[
            pl.BlockSpec((T, D), lambda i: (0, 0)),
            pl.BlockSpec((1, D, E + 1), lambda i: (hop, 0, 0)),
        ],
        out_specs=(
            pl.BlockSpec((E, CAP), lambda i: (0, 0)),
            pl.BlockSpec((T, K), lambda i: (0, 0)),
            pl.BlockSpec((T, K), lambda i: (0, 0)),
            pl.BlockSpec((T, 1), lambda i: (0, 0)),
        ),
        out_shape=(
            jax.ShapeDtypeStruct((E, CAP), jnp.int32),
            jax.ShapeDtypeStruct((T, K), jnp.int32),
            jax.ShapeDtypeStruct((T, K), jnp.float32),
            jax.ShapeDtypeStruct((T, 1), jnp.float32),
        ),
    )(h, w_router)


def _mlp_body(x_ref, w1_ref, w2_ref, y_ref):
    f = pl.program_id(1)
    hidden = jnp.dot(x_ref[0].astype(jnp.bfloat16),
                     w1_ref[0, 0].astype(jnp.bfloat16),
                     preferred_element_type=jnp.float32)
    hidden = jax.nn.gelu(hidden)
    y = jnp.dot(hidden.astype(jnp.bfloat16), w2_ref[0, 0].astype(jnp.bfloat16),
                preferred_element_type=jnp.float32)

    @pl.when(f == 0)
    def _():
        y_ref[0] = y

    @pl.when(f != 0)
    def _():
        y_ref[0] += y


def _mlp(hop, e_base, x_buf, w1, w2):
    n_e = x_buf.shape[0]
    ft = 4
    return pl.pallas_call(
        _mlp_body,
        grid=(n_e, ft),
        in_specs=[
            pl.BlockSpec((1, CAP, D), lambda e, f: (e, 0, 0)),
            pl.BlockSpec((1, 1, D, DFF // ft),
                         lambda e, f: (hop, e_base + e, 0, f)),
            pl.BlockSpec((1, 1, DFF // ft, D),
                         lambda e, f: (hop, e_base + e, f, 0)),
        ],
        out_specs=pl.BlockSpec((1, CAP, D), lambda e, f: (e, 0, 0)),
        out_shape=jax.ShapeDtypeStruct((n_e, CAP, D), jnp.float32),
    )(x_buf, w1, w2)


def _combine_body(h_ref, g_ref, wgt_ref, rho_ref, o_ref):
    h = h_ref[...]
    w0 = wgt_ref[:, 0:1]
    w1 = wgt_ref[:, 1:2]
    o_ref[...] = (
        h * (1.0 - rho_ref[...]) + w0 * g_ref[:, 0, :] + w1 * g_ref[:, 1, :]
    )


def _combine(h, g, wgt, rho):
    tt = 4
    return pl.pallas_call(
        _combine_body,
        grid=(tt,),
        in_specs=[
            pl.BlockSpec((T // tt, D), lambda i: (i, 0)),
            pl.BlockSpec((T // tt, K, D), lambda i: (i, 0, 0)),
            pl.BlockSpec((T // tt, K), lambda i: (i, 0)),
            pl.BlockSpec((T // tt, 1), lambda i: (i, 0)),
        ],
        out_specs=pl.BlockSpec((T // tt, D), lambda i: (i, 0)),
        out_shape=jax.ShapeDtypeStruct((T, D), jnp.float32),
    )(h, g, wgt, rho)


def _combine_rms_body(h_ref, g_ref, wgt_ref, rho_ref, wln_ref, o_ref):
    h = h_ref[...]
    w0 = wgt_ref[:, 0:1]
    w1 = wgt_ref[:, 1:2]
    hn = h * (1.0 - rho_ref[...]) + w0 * g_ref[:, 0, :] + w1 * g_ref[:, 1, :]
    rms = jnp.sqrt(jnp.mean(hn * hn, axis=1, keepdims=True) + EPS)
    o_ref[...] = ((hn / rms) * wln_ref[...]).astype(jnp.bfloat16)


def _combine_rms(h, g, wgt, rho, wln):
    tt = 4
    return pl.pallas_call(
        _combine_rms_body,
        grid=(tt,),
        in_specs=[
            pl.BlockSpec((T // tt, D), lambda i: (i, 0)),
            pl.BlockSpec((T // tt, K, D), lambda i: (i, 0, 0)),
            pl.BlockSpec((T // tt, K), lambda i: (i, 0)),
            pl.BlockSpec((T // tt, 1), lambda i: (i, 0)),
            pl.BlockSpec((1, D), lambda i: (0, 0)),
        ],
        out_specs=pl.BlockSpec((T // tt, D), lambda i: (i, 0)),
        out_shape=jax.ShapeDtypeStruct((T, D), jnp.bfloat16),
    )(h, g, wgt, rho, wln)


def _unembed_body(hn_ref, we_ref, o_ref):
    o_ref[...] = lax.dot_general(
        hn_ref[...],
        we_ref[...].astype(jnp.bfloat16),
        (((1,), (1,)), ((), ())),
        preferred_element_type=jnp.float32,
    )


def _unembed(hn, we):
    vt = 1280
    return pl.pallas_call(
        _unembed_body,
        grid=(VOCAB // vt,),
        in_specs=[
            pl.BlockSpec((T, D), lambda v: (0, 0)),
            pl.BlockSpec((vt, D), lambda v: (v, 0)),
        ],
        out_specs=pl.BlockSpec((T, vt), lambda v: (0, v)),
        out_shape=jax.ShapeDtypeStruct((T, VOCAB), jnp.float32),
    )(hn, we)


def kernel(ids_t, W_embed, w_ln, W_router, W1, W2):
    h = _sc_gather_rows(W_embed, ids_t.astype(jnp.int32), n_chunks=2)
    eh = E // 2
    for hop in range(HOPS):
        tok_idx, slot, wgt, rho = _routing(hop, h, W_router)
        xb0 = _sc_gather_rows(h, tok_idx[:eh].reshape(-1), n_chunks=2)
        xb1 = _sc_gather_rows(h, tok_idx[eh:].reshape(-1), n_chunks=2)
        y0 = _mlp(hop, 0, xb0.reshape(eh, CAP, D), W1, W2)
        y1 = _mlp(hop, eh, xb1.reshape(eh, CAP, D), W1, W2)
        y = jnp.concatenate([y0, y1], axis=0)
        g = _sc_gather_rows(y.reshape(E * CAP, D), slot.reshape(-1), n_chunks=4)
        if hop < HOPS - 1:
            h = _combine(h, g.reshape(T, K, D), wgt, rho)
        else:
            hn = _combine_rms(h, g.reshape(T, K, D), wgt, rho, w_ln.reshape(1, D))
    return _unembed(hn, W_embed)
